# baseline (speedup 1.0000x reference)
import functools

import jax
import jax.numpy as jnp
from jax import lax
from jax.experimental import pallas as pl
from jax.experimental.pallas import tpu as pltpu
from jax.experimental.pallas import tpu_sc as plsc

N = 10000
D = 128
R = 4
E = 320000
U = 128

NC = 2
NS = 16
L = 16

EPC = E // NC
EPW = E // (NC * NS)
K = 80
NCH = EPW // K
NBUF = 4

BLK_ROWS = 400
NBLK = N // BLK_ROWS
BLK_ITERS = (NBLK + NS - 1) // NS


def _matmul_body(x_ref, w_ref, y_ref):
    y_ref[0] = jnp.dot(x_ref[...], w_ref[0], preferred_element_type=jnp.float32)


def _matmul(x2d, W):
    return pl.pallas_call(
        _matmul_body,
        grid=(R,),
        in_specs=[
            pl.BlockSpec((N, D), lambda r: (0, 0)),
            pl.BlockSpec((1, D, U), lambda r: (r, 0, 0)),
        ],
        out_specs=pl.BlockSpec((1, N, U), lambda r: (r, 0, 0)),
        out_shape=jax.ShapeDtypeStruct((R, N, U), jnp.float32),
    )(x2d, W)


def _lane_bcast(v16, i):
    idx = jnp.full((L, 1), i, jnp.int32)
    dn = lax.GatherDimensionNumbers(
        offset_dims=(), collapsed_slice_dims=(0,), start_index_map=(0,))
    return lax.gather(v16, idx, dn, (1,),
                      mode=lax.GatherScatterMode.PROMISE_IN_BOUNDS)


NIB = 8


def _sc_body(y_hbm, ei_hbm, vals_hbm, zeros_hbm, out_hbm, *scr):
    rows_b = scr[0:NIB]
    cols_b = scr[NIB:2 * NIB]
    vals_b = scr[2 * NIB:3 * NIB]
    bufs = scr[3 * NIB:3 * NIB + NBUF]
    isems = scr[3 * NIB + NBUF:4 * NIB + NBUF]
    gsems = scr[4 * NIB + NBUF:4 * NIB + 2 * NBUF]
    ssems = scr[4 * NIB + 2 * NBUF:4 * NIB + 3 * NBUF]
    acc_sh = scr[4 * NIB + 3 * NBUF]
    cid = lax.axis_index("c")
    sid = lax.axis_index("s")

    def init_body(j, _):
        blk = j * NS + sid

        @pl.when(blk < NBLK)
        def _():
            sl = pl.ds(blk * BLK_ROWS, BLK_ROWS)
            pltpu.sync_copy(zeros_hbm.at[sl], acc_sh.at[sl])
        return 0

    lax.fori_loop(0, BLK_ITERS, init_body, 0, unroll=True)
    plsc.subcore_barrier()

    def issue_idx(base, vbase, c, b):
        off = base + c * K
        voff = vbase + c * K
        pltpu.async_copy(ei_hbm.at[pl.ds(off, K)], rows_b[b], isems[b])
        pltpu.async_copy(ei_hbm.at[pl.ds(off + E, K)], cols_b[b], isems[b])
        pltpu.async_copy(vals_hbm.at[pl.ds(voff, K)], vals_b[b], isems[b])

    def wait_idx(base, vbase, c, b):
        off = base + c * K
        voff = vbase + c * K
        pltpu.make_async_copy(ei_hbm.at[pl.ds(off, K)], rows_b[b],
                              isems[b]).wait()
        pltpu.make_async_copy(ei_hbm.at[pl.ds(off + E, K)], cols_b[b],
                              isems[b]).wait()
        pltpu.make_async_copy(vals_hbm.at[pl.ds(voff, K)], vals_b[b],
                              isems[b]).wait()

    def issue_gather(r, g, b):
        pltpu.async_copy(y_hbm.at[r].at[cols_b[b]], bufs[g], gsems[g])

    def wait_gather(r, g, b):
        pltpu.make_async_copy(y_hbm.at[r].at[cols_b[b]], bufs[g],
                              gsems[g]).wait()

    def wait_scatter(g, b):
        pltpu.make_async_copy(bufs[g], acc_sh.at[rows_b[b]], ssems[g]).wait()

    def process(g, b):
        buf = bufs[g]
        vals_v = vals_b[b]

        def group_body(j, _):
            v16 = vals_v[pl.ds(j * L, L)]
            for i in range(L):
                vsp = _lane_bcast(v16, i)
                row = j * L + i
                for cc in range(U // L):
                    sl = pl.ds(cc * L, L)
                    buf[row, sl] = buf[row, sl] * vsp
            return 0

        lax.fori_loop(0, K // L, group_body, 0)
        pltpu.async_copy(buf, acc_sh.at[rows_b[b]], ssems[g], add=True)

    def rel_body(r, _):
        base = r * (2 * E) + cid * EPC + sid * EPW
        vbase = r * E + cid * EPC + sid * EPW
        issue_idx(base, vbase, 0, 0)
        issue_idx(base, vbase, 1, 1)
        issue_idx(base, vbase, 2, 2)
        wait_idx(base, vbase, 0, 0)
        issue_gather(r, 0, 0)
        wait_idx(base, vbase, 1, 1)
        issue_gather(r, 1, 1)

        def oct_body(q, _):
            for p in range(NIB):
                c = q * NIB + p

                @pl.when(c < NCH)
                def _(c=c, p=p):
                    g = p % NBUF

                    @pl.when(c >= 2)
                    def _():
                        wait_scatter((p + 2) % NBUF, (p + 6) % NIB)

                    @pl.when(c + 3 < NCH)
                    def _():
                        issue_idx(base, vbase, c + 3, (p + 3) % NIB)

                    @pl.when(c + 2 < NCH)
                    def _():
                        wait_idx(base, vbase, c + 2, (p + 2) % NIB)
                        issue_gather(r, (p + 2) % NBUF, (p + 2) % NIB)

                    wait_gather(r, g, p)
                    process(g, p)
            return 0

        lax.fori_loop(0, (NCH + NIB - 1) // NIB, oct_body, 0)

        wait_scatter((NCH - 2) % NBUF, (NCH - 2) % NIB)
        wait_scatter((NCH - 1) % NBUF, (NCH - 1) % NIB)
        return 0

    lax.fori_loop(0, R, rel_body, 0)

    plsc.subcore_barrier()

    def wb_body(j, _):
        blk = j * NS + sid

        @pl.when(blk < NBLK)
        def _():
            sl = pl.ds(blk * BLK_ROWS, BLK_ROWS)
            pltpu.sync_copy(acc_sh.at[sl], out_hbm.at[cid].at[sl])
        return 0

    lax.fori_loop(0, BLK_ITERS, wb_body, 0, unroll=True)


def _sc_scatter(y, ei_flat, vals_flat, zeros):
    fn = pl.kernel(
        _sc_body,
        out_type=jax.ShapeDtypeStruct((NC, N, U), jnp.float32),
        mesh=plsc.VectorSubcoreMesh(core_axis_name="c", subcore_axis_name="s"),
        scratch_types=(
            [pltpu.VMEM((K,), jnp.int32) for _ in range(NIB)]
            + [pltpu.VMEM((K,), jnp.int32) for _ in range(NIB)]
            + [pltpu.VMEM((K,), jnp.float32) for _ in range(NIB)]
            + [pltpu.VMEM((K, U), jnp.float32) for _ in range(NBUF)]
            + [pltpu.SemaphoreType.DMA for _ in range(NIB)]
            + [pltpu.SemaphoreType.DMA for _ in range(2 * NBUF)]
            + [pltpu.VMEM_SHARED((N, U), jnp.float32)]
        ),
    )
    return fn(y, ei_flat, vals_flat, zeros)


def _epilogue_body(p_ref, b_ref, o_ref):
    o_ref[...] = jnp.maximum(p_ref[0] + p_ref[1] + b_ref[...], 0.0)


def _epilogue(partials, b2d):
    blk = 2000
    return pl.pallas_call(
        _epilogue_body,
        grid=(N // blk,),
        in_specs=[
            pl.BlockSpec((NC, blk, U), lambda i: (0, i, 0)),
            pl.BlockSpec((1, U), lambda i: (0, 0)),
        ],
        out_specs=pl.BlockSpec((blk, U), lambda i: (i, 0)),
        out_shape=jax.ShapeDtypeStruct((N, U), jnp.float32),
    )(partials, b2d)


@jax.jit
def kernel(x, edge_index, edge_values, W, b):
    x2d = x[0]
    ei_flat = edge_index.astype(jnp.int32).reshape(-1)
    vals_flat = edge_values.astype(jnp.float32).reshape(-1)
    y = _matmul(x2d, W)
    partials = _sc_scatter(y, ei_flat, vals_flat,
                           jnp.zeros((N, U), jnp.float32))
    out = _epilogue(partials, b.reshape(1, U))
    return out.reshape(1, N, U)


# speedup vs baseline: 1.0144x; 1.0144x over previous
import functools

import jax
import jax.numpy as jnp
from jax import lax
from jax.experimental import pallas as pl
from jax.experimental.pallas import tpu as pltpu
from jax.experimental.pallas import tpu_sc as plsc

N = 10000
D = 128
R = 4
E = 320000
U = 128

NC = 2
NS = 16
L = 16

EPC = E // NC
EPW = E // (NC * NS)
K = 80
NCH = EPW // K
NBUF = 4

BLK_ROWS = 400
NBLK = N // BLK_ROWS
BLK_ITERS = (NBLK + NS - 1) // NS


def _matmul_body(x_ref, w_ref, y_ref):
    y_ref[0] = jnp.dot(x_ref[...], w_ref[0], preferred_element_type=jnp.float32)


def _matmul(x2d, W):
    return pl.pallas_call(
        _matmul_body,
        grid=(R,),
        in_specs=[
            pl.BlockSpec((N, D), lambda r: (0, 0)),
            pl.BlockSpec((1, D, U), lambda r: (r, 0, 0)),
        ],
        out_specs=pl.BlockSpec((1, N, U), lambda r: (r, 0, 0)),
        out_shape=jax.ShapeDtypeStruct((R, N, U), jnp.float32),
    )(x2d, W)


def _lane_bcast(v16, i):
    idx = jnp.full((L, 1), i, jnp.int32)
    dn = lax.GatherDimensionNumbers(
        offset_dims=(), collapsed_slice_dims=(0,), start_index_map=(0,))
    return lax.gather(v16, idx, dn, (1,),
                      mode=lax.GatherScatterMode.PROMISE_IN_BOUNDS)


NIB = 8


def _sc_body(y_hbm, rows_hbm, cols_hbm, vals_hbm, zeros_hbm, out_hbm, *scr):
    rows_b = scr[0:NIB]
    cols_b = scr[NIB:2 * NIB]
    vals_b = scr[2 * NIB:3 * NIB]
    bufs = scr[3 * NIB:3 * NIB + NBUF]
    isems = scr[3 * NIB + NBUF:4 * NIB + NBUF]
    gsems = scr[4 * NIB + NBUF:4 * NIB + 2 * NBUF]
    ssems = scr[4 * NIB + 2 * NBUF:4 * NIB + 3 * NBUF]
    acc_sh = scr[4 * NIB + 3 * NBUF]
    cid = lax.axis_index("c")
    sid = lax.axis_index("s")

    def init_body(j, _):
        blk = j * NS + sid

        @pl.when(blk < NBLK)
        def _():
            sl = pl.ds(blk * BLK_ROWS, BLK_ROWS)
            pltpu.sync_copy(zeros_hbm.at[sl], acc_sh.at[sl])
        return 0

    lax.fori_loop(0, BLK_ITERS, init_body, 0, unroll=True)
    plsc.subcore_barrier()

    def issue_idx(base, c, b):
        off = base + c * K
        pltpu.async_copy(rows_hbm.at[pl.ds(off, K)], rows_b[b], isems[b])
        pltpu.async_copy(cols_hbm.at[pl.ds(off, K)], cols_b[b], isems[b])
        pltpu.async_copy(vals_hbm.at[pl.ds(off, K)], vals_b[b], isems[b])

    def wait_idx(base, c, b):
        off = base + c * K
        pltpu.make_async_copy(rows_hbm.at[pl.ds(off, K)], rows_b[b],
                              isems[b]).wait()
        pltpu.make_async_copy(cols_hbm.at[pl.ds(off, K)], cols_b[b],
                              isems[b]).wait()
        pltpu.make_async_copy(vals_hbm.at[pl.ds(off, K)], vals_b[b],
                              isems[b]).wait()

    def issue_gather(r, g, b):
        pltpu.async_copy(y_hbm.at[r].at[cols_b[b]], bufs[g], gsems[g])

    def wait_gather(r, g, b):
        pltpu.make_async_copy(y_hbm.at[r].at[cols_b[b]], bufs[g],
                              gsems[g]).wait()

    def wait_scatter(g, b):
        pltpu.make_async_copy(bufs[g], acc_sh.at[rows_b[b]], ssems[g]).wait()

    def process(g, b):
        buf = bufs[g]
        vals_v = vals_b[b]

        def group_body(j, _):
            v16 = vals_v[pl.ds(j * L, L)]
            for i in range(L):
                vsp = _lane_bcast(v16, i)
                row = j * L + i
                for cc in range(U // L):
                    sl = pl.ds(cc * L, L)
                    buf[row, sl] = buf[row, sl] * vsp
            return 0

        lax.fori_loop(0, K // L, group_body, 0)
        pltpu.async_copy(buf, acc_sh.at[rows_b[b]], ssems[g], add=True)

    def rel_body(r, _):
        base = r * E + cid * EPC + sid * EPW
        issue_idx(base, 0, 0)
        issue_idx(base, 1, 1)
        issue_idx(base, 2, 2)
        wait_idx(base, 0, 0)
        issue_gather(r, 0, 0)
        wait_idx(base, 1, 1)
        issue_gather(r, 1, 1)

        def oct_body(q, _):
            for p in range(NIB):
                c = q * NIB + p

                @pl.when(c < NCH)
                def _(c=c, p=p):
                    g = p % NBUF

                    @pl.when(c >= 2)
                    def _():
                        wait_scatter((p + 2) % NBUF, (p + 6) % NIB)

                    @pl.when(c + 3 < NCH)
                    def _():
                        issue_idx(base, c + 3, (p + 3) % NIB)

                    @pl.when(c + 2 < NCH)
                    def _():
                        wait_idx(base, c + 2, (p + 2) % NIB)
                        issue_gather(r, (p + 2) % NBUF, (p + 2) % NIB)

                    wait_gather(r, g, p)
                    process(g, p)
            return 0

        lax.fori_loop(0, (NCH + NIB - 1) // NIB, oct_body, 0)

        wait_scatter((NCH - 2) % NBUF, (NCH - 2) % NIB)
        wait_scatter((NCH - 1) % NBUF, (NCH - 1) % NIB)
        return 0

    lax.fori_loop(0, R, rel_body, 0)

    plsc.subcore_barrier()

    def wb_body(j, _):
        blk = j * NS + sid

        @pl.when(blk < NBLK)
        def _():
            sl = pl.ds(blk * BLK_ROWS, BLK_ROWS)
            pltpu.sync_copy(acc_sh.at[sl], out_hbm.at[cid].at[sl])
        return 0

    lax.fori_loop(0, BLK_ITERS, wb_body, 0, unroll=True)


def _sc_scatter(y, rows_flat, cols_flat, vals_flat, zeros):
    fn = pl.kernel(
        _sc_body,
        out_type=jax.ShapeDtypeStruct((NC, N, U), jnp.float32),
        mesh=plsc.VectorSubcoreMesh(core_axis_name="c", subcore_axis_name="s"),
        scratch_types=(
            [pltpu.VMEM((K,), jnp.int32) for _ in range(NIB)]
            + [pltpu.VMEM((K,), jnp.int32) for _ in range(NIB)]
            + [pltpu.VMEM((K,), jnp.float32) for _ in range(NIB)]
            + [pltpu.VMEM((K, U), jnp.float32) for _ in range(NBUF)]
            + [pltpu.SemaphoreType.DMA for _ in range(NIB)]
            + [pltpu.SemaphoreType.DMA for _ in range(2 * NBUF)]
            + [pltpu.VMEM_SHARED((N, U), jnp.float32)]
        ),
    )
    return fn(y, rows_flat, cols_flat, vals_flat, zeros)


def _epilogue_body(p_ref, b_ref, o_ref):
    o_ref[...] = jnp.maximum(p_ref[0] + p_ref[1] + b_ref[...], 0.0)


def _epilogue(partials, b2d):
    blk = 2000
    return pl.pallas_call(
        _epilogue_body,
        grid=(N // blk,),
        in_specs=[
            pl.BlockSpec((NC, blk, U), lambda i: (0, i, 0)),
            pl.BlockSpec((1, U), lambda i: (0, 0)),
        ],
        out_specs=pl.BlockSpec((blk, U), lambda i: (i, 0)),
        out_shape=jax.ShapeDtypeStruct((N, U), jnp.float32),
    )(partials, b2d)


@jax.jit
def kernel(x, edge_index, edge_values, W, b):
    x2d = x[0]
    ei = edge_index.astype(jnp.int32)
    rows_flat = ei[:, 0, :].reshape(-1)
    cols_flat = ei[:, 1, :].reshape(-1)
    vals_flat = edge_values.astype(jnp.float32).reshape(-1)
    y = _matmul(x2d, W)
    partials = _sc_scatter(y, rows_flat, cols_flat, vals_flat,
                           jnp.zeros((N, U), jnp.float32))
    out = _epilogue(partials, b.reshape(1, U))
    return out.reshape(1, N, U)
